# baseline (speedup 1.0000x reference)
import jax
import jax.numpy as jnp
from jax.experimental import pallas as pl
from jax.experimental.pallas import tpu as pltpu

_N = 471
_TOPK = 50
_RHO = 1.0
_W = 0.01
_LAMDA = 0.1
_MU = 0.01


def _body(theta_row_ref, theta_col_ref, z_ref, u_ref, A_ref, qt_ref,
          wlin_ref, blin_ref, out_ref):
    theta_row = theta_row_ref[...]
    theta_col = theta_col_ref[...]
    z = z_ref[...]
    u = u_ref[...]
    A = A_ref[...]
    qt = qt_ref[...]
    wlin = wlin_ref[...]
    blin = blin_ref[...]

    f32 = jnp.float32
    qtheta_row = jax.lax.dot_general(
        theta_row, qt, (((1,), (0,)), ((), ())), preferred_element_type=f32)
    qtheta_col = jax.lax.dot_general(
        qt, theta_col, (((0,), (0,)), ((), ())), preferred_element_type=f32)
    b_row = _W + (1.0 / _N) * qtheta_row
    b_col = _W + (1.0 / _N) * qtheta_col

    ones_row = jnp.ones((1, 4096), dtype=f32)
    a_mean = (1.0 / 4096.0) * jax.lax.dot_general(
        ones_row, A, (((1,), (0,)), ((), ())), preferred_element_type=f32)
    w2_row = jax.lax.dot_general(
        a_mean, wlin, (((1,), (1,)), ((), ())),
        preferred_element_type=f32) + blin

    zsum = jnp.sum(z)
    gsum = 2.0 * _LAMDA * (zsum - 1.0)

    def z_new(zv, uv, bv, w2v):
        grad = w2v + _RHO * (zv - bv) + uv + gsum \
            + 2.0 * _LAMDA * jnp.minimum(0.0, zv)
        return jnp.maximum(zv - _MU * grad, 0.0)

    zn_row = z_new(z, u, b_row, w2_row)
    z_col = jnp.transpose(z)
    u_col = jnp.transpose(u)
    w2_col = jnp.transpose(w2_row)
    zn_col = z_new(z_col, u_col, b_col, w2_col)

    gt = (zn_col > zn_row).astype(f32)
    rank = jnp.sum(gt, axis=0, keepdims=True)
    out_ref[...] = jnp.where(rank < float(_TOPK), zn_row, 0.0)


def kernel(theta, z, u, A, q_t, W_lin, b_lin):
    theta_row = theta.reshape(1, 4096)
    theta_col = theta.reshape(4096, 1)
    zr = z.reshape(1, _N)
    ur = u.reshape(1, _N)
    br = b_lin.reshape(1, _N)
    out = pl.pallas_call(
        _body,
        out_shape=jax.ShapeDtypeStruct((1, _N), jnp.float32),
    )(theta_row, theta_col, zr, ur, A, q_t, W_lin, br)
    return (out.reshape(_N), q_t)
